# baseline (speedup 1.0000x reference)
import functools

import jax
import jax.numpy as jnp
from jax import lax
from jax.experimental import pallas as pl
from jax.experimental.pallas import tpu as pltpu
from jax.experimental.pallas import tpu_sc as plsc

N = 10000
E = 320000
NUM_GRAPHS = 64
IN_CH = 128
HID = 16
HEADS = 8

NPAD = 10240
NW = 32
EPAD = 344064
DEPTH = 3
IDEPTH = 6
BLK = 1024
GRID = NPAD // BLK


def _mm2_body(x_ref, a_ref, b_ref, o1_ref, o2_ref):
    x = x_ref[...]
    o1_ref[...] = jnp.dot(x, a_ref[...], preferred_element_type=jnp.float32)
    o2_ref[...] = jnp.dot(x, b_ref[...], preferred_element_type=jnp.float32)


def _mm2(x, a, b):
    m = x.shape[0]
    return pl.pallas_call(
        _mm2_body,
        grid=(m // BLK,),
        in_specs=[
            pl.BlockSpec((BLK, x.shape[1]), lambda i: (i, 0)),
            pl.BlockSpec(a.shape, lambda i: (0, 0)),
            pl.BlockSpec(b.shape, lambda i: (0, 0)),
        ],
        out_specs=[
            pl.BlockSpec((BLK, a.shape[1]), lambda i: (i, 0)),
            pl.BlockSpec((BLK, b.shape[1]), lambda i: (i, 0)),
        ],
        out_shape=[
            jax.ShapeDtypeStruct((m, a.shape[1]), jnp.float32),
            jax.ShapeDtypeStruct((m, b.shape[1]), jnp.float32),
        ],
    )(x, a, b)


def _norm_mm2_body(p_ref, e_ref, bias_ref, a_ref, b_ref, o1_ref, o2_ref):
    acc = p_ref[0] + p_ref[1]
    msg = acc[:, :IN_CH]
    den = acc[:, IN_CH:IN_CH + HEADS]
    den_exp = jnp.dot(den, e_ref[...], preferred_element_type=jnp.float32)
    x2 = jnp.maximum(msg / (den_exp + 1e-16) + bias_ref[...], 0.0)
    o1_ref[...] = jnp.dot(x2, a_ref[...], preferred_element_type=jnp.float32)
    o2_ref[...] = jnp.dot(x2, b_ref[...], preferred_element_type=jnp.float32)


def _norm_mm2(p, e, bias, a, b):
    return pl.pallas_call(
        _norm_mm2_body,
        grid=(GRID,),
        in_specs=[
            pl.BlockSpec((2, BLK, IN_CH + HID), lambda i: (0, i, 0)),
            pl.BlockSpec(e.shape, lambda i: (0, 0)),
            pl.BlockSpec(bias.shape, lambda i: (0, 0)),
            pl.BlockSpec(a.shape, lambda i: (0, 0)),
            pl.BlockSpec(b.shape, lambda i: (0, 0)),
        ],
        out_specs=[
            pl.BlockSpec((BLK, a.shape[1]), lambda i: (i, 0)),
            pl.BlockSpec((BLK, b.shape[1]), lambda i: (i, 0)),
        ],
        out_shape=[
            jax.ShapeDtypeStruct((NPAD, a.shape[1]), jnp.float32),
            jax.ShapeDtypeStruct((NPAD, b.shape[1]), jnp.float32),
        ],
    )(p, e, bias, a, b)


def _pool_body(p_ref, batch_ref, bias_ref, o_ref, sums, cnts):
    i = pl.program_id(0)
    acc = p_ref[0] + p_ref[1]
    o2 = acc[:, :HID] / (acc[:, HID:HID + 1] + 1e-16) + bias_ref[...]
    bv = batch_ref[0, 0, :]
    gid = lax.broadcasted_iota(jnp.int32, (BLK, NUM_GRAPHS), 1)
    oh = (bv[:, None] == gid).astype(jnp.float32)
    s = lax.dot_general(oh, o2, (((0,), (0,)), ((), ())),
                        preferred_element_type=jnp.float32)
    c = lax.dot_general(oh, jnp.ones((BLK, 1), jnp.float32),
                        (((0,), (0,)), ((), ())),
                        preferred_element_type=jnp.float32)

    @pl.when(i == 0)
    def _():
        sums[...] = s
        cnts[...] = c

    @pl.when(i > 0)
    def _():
        sums[...] += s
        cnts[...] += c

    @pl.when(i == GRID - 1)
    def _():
        o_ref[...] = sums[...] / jnp.maximum(cnts[...], 1.0)


def _pool(p, batch3, bias):
    return pl.pallas_call(
        _pool_body,
        grid=(GRID,),
        in_specs=[
            pl.BlockSpec((2, BLK, 2 * HID), lambda i: (0, i, 0)),
            pl.BlockSpec((1, 1, BLK), lambda i: (i, 0, 0)),
            pl.BlockSpec(bias.shape, lambda i: (0, 0)),
        ],
        out_specs=pl.BlockSpec((NUM_GRAPHS, HID), lambda i: (0, 0)),
        out_shape=jax.ShapeDtypeStruct((NUM_GRAPHS, HID), jnp.float32),
        scratch_shapes=[
            pltpu.VMEM((NUM_GRAPHS, HID), jnp.float32),
            pltpu.VMEM((NUM_GRAPHS, 1), jnp.float32),
        ],
    )(p, batch3, bias)


def _make_edge_pass(dh, ch):
    R = dh + 16
    nchunk = EPAD // (NW * ch)
    rows_per_tile = NPAD // 16
    nzcopy = rows_per_tile // ch
    mesh = plsc.VectorSubcoreMesh(core_axis_name="c", subcore_axis_name="s")

    @functools.partial(
        pl.kernel, mesh=mesh,
        compiler_params=pltpu.CompilerParams(use_tc_tiling_on_sc=False),
        out_type=jax.ShapeDtypeStruct((2, NPAD, R), jnp.float32),
        scratch_types=[
            [pltpu.VMEM((ch,), jnp.int32)] * IDEPTH,
            [pltpu.VMEM((ch,), jnp.int32)] * IDEPTH,
            [pltpu.VMEM((ch, R), jnp.float32)] * DEPTH,
            [pltpu.VMEM((ch, 16), jnp.float32)] * DEPTH,
            pltpu.VMEM_SHARED((NPAD, R), jnp.float32),
            [pltpu.SemaphoreType.DMA] * IDEPTH,
            [pltpu.SemaphoreType.DMA] * DEPTH,
            [pltpu.SemaphoreType.DMA] * DEPTH,
            [pltpu.SemaphoreType.DMA] * DEPTH,
        ],
    )
    def edge_pass(htab, ttab, src_hbm, dst_hbm, out_hbm,
                  src_idx, dst_idx, bufs, tbufs, accum, semi, semg, semt, sems):
        cid = lax.axis_index("c")
        sid = lax.axis_index("s")
        wid = cid * 16 + sid
        row0 = sid * rows_per_tile
        cbase = wid * nchunk

        def zrow(r, carry):
            for j in range(R // 16):
                bufs[0][r, pl.ds(j * 16, 16)] = jnp.zeros((16,), jnp.float32)
            return carry
        lax.fori_loop(0, ch, zrow, 0)
        for j in range(nzcopy):
            pltpu.sync_copy(bufs[0], accum.at[pl.ds(row0 + j * ch, ch)])
        plsc.subcore_barrier()

        def issue_idx(c, bi):
            pltpu.async_copy(src_hbm.at[cbase + c], src_idx[bi], semi[bi])
            pltpu.async_copy(dst_hbm.at[cbase + c], dst_idx[bi], semi[bi])

        def wait_idx(bi):
            pltpu.make_async_copy(src_hbm.at[0], src_idx[bi], semi[bi]).wait()
            pltpu.make_async_copy(dst_hbm.at[0], dst_idx[bi], semi[bi]).wait()

        def issue_gather(b, bi):
            pltpu.async_copy(htab.at[src_idx[bi]], bufs[b], semg[b])
            pltpu.async_copy(ttab.at[dst_idx[bi]], tbufs[b], semt[b])

        def wait_gather(b):
            pltpu.make_async_copy(htab.at[src_idx[0]], bufs[b], semg[b]).wait()
            pltpu.make_async_copy(ttab.at[dst_idx[0]], tbufs[b], semt[b]).wait()

        def issue_scatter(b, bi):
            pltpu.async_copy(bufs[b], accum.at[dst_idx[bi]], sems[b], add=True)

        def wait_scatter(b):
            pltpu.make_async_copy(bufs[b], accum.at[dst_idx[0]], sems[b]).wait()

        def compute(b):
            buf = bufs[b]
            tbuf = tbufs[b]

            def edge(e2, c2):
                for u in range(2):
                    e = e2 * 2 + u
                    s = buf[e, pl.ds(dh, 16)] + tbuf[e, :]
                    s = jnp.maximum(s, s * 0.2)
                    w = jnp.exp(s)
                    buf[e, pl.ds(dh, 16)] = w
                    for hd in range(dh // 16):
                        buf[e, pl.ds(hd * 16, 16)] = (
                            buf[e, pl.ds(hd * 16, 16)] * w[hd])
                return c2
            lax.fori_loop(0, ch // 2, edge, 0)

        issue_idx(0, 0)
        issue_idx(1, 1)
        issue_idx(2, 2)
        wait_idx(0)
        issue_gather(0, 0)
        wait_idx(1)
        issue_gather(1, 1)

        def outer(g, carry):
            for u in range(IDEPTH):
                c = g * IDEPTH + u
                b = u % DEPTH
                wait_gather(b)
                compute(b)
                issue_scatter(b, u)

                @pl.when(c >= 1)
                def _():
                    wait_scatter((b + 2) % DEPTH)

                @pl.when(c + 2 < nchunk)
                def _():
                    wait_idx((u + 2) % IDEPTH)
                    issue_gather((b + 2) % DEPTH, (u + 2) % IDEPTH)

                @pl.when(c + 3 < nchunk)
                def _():
                    issue_idx(c + 3, (u + 3) % IDEPTH)
            return carry
        lax.fori_loop(0, nchunk // IDEPTH, outer, 0)

        wait_scatter((nchunk - 1) % DEPTH)
        plsc.subcore_barrier()
        pltpu.sync_copy(accum.at[pl.ds(row0, rows_per_tile)],
                        out_hbm.at[cid, pl.ds(row0, rows_per_tile)])

    return edge_pass


CH1 = 64
CH2 = 128
_edge_pass_1 = _make_edge_pass(IN_CH, CH1)
_edge_pass_2 = _make_edge_pass(HID, CH2)


def kernel(x, edge_index, edge_attr, batch, W1, a_src1, a_dst1, b1,
           W2, a_src2, a_dst2, b2):
    del edge_attr
    W1r = W1.reshape(IN_CH, HEADS, HID)
    ws1 = jnp.einsum('ihc,hc->ih', W1r, a_src1)
    wd1 = jnp.einsum('ihc,hc->ih', W1r, a_dst1)
    B1a = jnp.concatenate([W1, ws1, ws1], axis=1)
    B1b = jnp.concatenate([wd1, wd1], axis=1)
    ws2 = (W2 @ a_src2[0])[:, None]
    wd2 = (W2 @ a_dst2[0])[:, None]
    B2a = jnp.concatenate([W2, jnp.tile(ws2, (1, HID))], axis=1)
    B2b = jnp.tile(wd2, (1, HID))

    xp = jnp.zeros((NPAD, IN_CH), jnp.float32).at[:N].set(x)
    idx_dtype = edge_index.dtype
    loop = jnp.arange(N, dtype=idx_dtype)
    fake = jnp.full((EPAD - E - N,), N, dtype=idx_dtype)
    src = jnp.concatenate([edge_index[0], loop, fake]).astype(jnp.int32)
    dst = jnp.concatenate([edge_index[1], loop, fake]).astype(jnp.int32)

    hrow = lax.broadcasted_iota(jnp.int32, (HEADS, IN_CH), 0)
    hcol = lax.broadcasted_iota(jnp.int32, (HEADS, IN_CH), 1) // HID
    e_exp = (hrow == hcol).astype(jnp.float32)

    batch_pad = jnp.concatenate(
        [batch.astype(jnp.int32),
         jnp.full((NPAD - N,), NUM_GRAPHS, jnp.int32)]).reshape(GRID, 1, BLK)

    h1, t1 = _mm2(xp, B1a, B1b)
    p1 = _edge_pass_1(h1, t1, src.reshape(-1, CH1), dst.reshape(-1, CH1))
    h2, t2 = _norm_mm2(p1, e_exp, b1.reshape(1, IN_CH), B2a, B2b)
    p2 = _edge_pass_2(h2, t2, src.reshape(-1, CH2), dst.reshape(-1, CH2))
    return _pool(p2, batch_pad, b2.reshape(1, HID))


# speedup vs baseline: 1.1956x; 1.1956x over previous
import functools

import jax
import jax.numpy as jnp
from jax import lax
from jax.experimental import pallas as pl
from jax.experimental.pallas import tpu as pltpu
from jax.experimental.pallas import tpu_sc as plsc

N = 10000
E = 320000
NUM_GRAPHS = 64
IN_CH = 128
HID = 16
HEADS = 8

NPAD = 10240
NW = 32
EPAD = 344064
DEPTH = 3
IDEPTH = 6
BLK = 1024
GRID = NPAD // BLK


def _mm_l1_body(x_ref, a_ref, b_ref, oh_ref, ot_ref):
    g = pl.program_id(0)
    x = x_ref[...]
    oh_ref[0] = jnp.dot(x, a_ref[0], preferred_element_type=jnp.float32)

    @pl.when(g == 0)
    def _():
        ot_ref[...] = jnp.dot(x, b_ref[...], preferred_element_type=jnp.float32)


def _mm_l1(x, a, b):
    return pl.pallas_call(
        _mm_l1_body,
        grid=(2, GRID),
        in_specs=[
            pl.BlockSpec((BLK, x.shape[1]), lambda g, i: (i, 0)),
            pl.BlockSpec((1,) + a.shape[1:], lambda g, i: (g, 0, 0)),
            pl.BlockSpec(b.shape, lambda g, i: (0, 0)),
        ],
        out_specs=[
            pl.BlockSpec((1, BLK, a.shape[2]), lambda g, i: (g, i, 0)),
            pl.BlockSpec((BLK, b.shape[1]), lambda g, i: (i, 0)),
        ],
        out_shape=[
            jax.ShapeDtypeStruct((2, NPAD, a.shape[2]), jnp.float32),
            jax.ShapeDtypeStruct((NPAD, b.shape[1]), jnp.float32),
        ],
    )(x, a, b)


def _norm_mm2_body(p_ref, bias_ref, a_ref, b_ref, o1_ref, o2_ref):
    hh = IN_CH // 2
    e4r = lax.broadcasted_iota(jnp.int32, (4, hh), 0)
    e4c = lax.broadcasted_iota(jnp.int32, (4, hh), 1) // HID
    e4 = (e4r == e4c).astype(jnp.float32)
    xa = p_ref[0, :, :hh] / (
        jnp.dot(p_ref[0, :, hh:hh + 4], e4,
                preferred_element_type=jnp.float32) + 1e-16)
    xb = p_ref[1, :, :hh] / (
        jnp.dot(p_ref[1, :, hh:hh + 4], e4,
                preferred_element_type=jnp.float32) + 1e-16)
    x2 = jnp.maximum(
        jnp.concatenate([xa, xb], axis=1) + bias_ref[...], 0.0)
    o1_ref[...] = jnp.dot(x2, a_ref[...], preferred_element_type=jnp.float32)
    o2_ref[...] = jnp.dot(x2, b_ref[...], preferred_element_type=jnp.float32)


def _norm_mm2(p, bias, a, b):
    hw = p.shape[2]
    return pl.pallas_call(
        _norm_mm2_body,
        grid=(GRID,),
        in_specs=[
            pl.BlockSpec((2, BLK, hw), lambda i: (0, i, 0)),
            pl.BlockSpec(bias.shape, lambda i: (0, 0)),
            pl.BlockSpec(a.shape, lambda i: (0, 0)),
            pl.BlockSpec(b.shape, lambda i: (0, 0)),
        ],
        out_specs=[
            pl.BlockSpec((BLK, a.shape[1]), lambda i: (i, 0)),
            pl.BlockSpec((BLK, b.shape[1]), lambda i: (i, 0)),
        ],
        out_shape=[
            jax.ShapeDtypeStruct((NPAD, a.shape[1]), jnp.float32),
            jax.ShapeDtypeStruct((NPAD, b.shape[1]), jnp.float32),
        ],
    )(p, bias, a, b)


def _pool_body(p_ref, batch_ref, bias_ref, o_ref, sums, cnts):
    i = pl.program_id(0)
    acc = p_ref[0] + p_ref[1]
    o2 = acc[:, :HID] / (acc[:, HID:HID + 1] + 1e-16) + bias_ref[...]
    bv = batch_ref[0, 0, :]
    gid = lax.broadcasted_iota(jnp.int32, (BLK, NUM_GRAPHS), 1)
    oh = (bv[:, None] == gid).astype(jnp.float32)
    s = lax.dot_general(oh, o2, (((0,), (0,)), ((), ())),
                        preferred_element_type=jnp.float32)
    c = lax.dot_general(oh, jnp.ones((BLK, 1), jnp.float32),
                        (((0,), (0,)), ((), ())),
                        preferred_element_type=jnp.float32)

    @pl.when(i == 0)
    def _():
        sums[...] = s
        cnts[...] = c

    @pl.when(i > 0)
    def _():
        sums[...] += s
        cnts[...] += c

    @pl.when(i == GRID - 1)
    def _():
        o_ref[...] = sums[...] / jnp.maximum(cnts[...], 1.0)


def _pool(p, batch3, bias):
    return pl.pallas_call(
        _pool_body,
        grid=(GRID,),
        in_specs=[
            pl.BlockSpec((2, BLK, 2 * HID), lambda i: (0, i, 0)),
            pl.BlockSpec((1, 1, BLK), lambda i: (i, 0, 0)),
            pl.BlockSpec(bias.shape, lambda i: (0, 0)),
        ],
        out_specs=pl.BlockSpec((NUM_GRAPHS, HID), lambda i: (0, 0)),
        out_shape=jax.ShapeDtypeStruct((NUM_GRAPHS, HID), jnp.float32),
        scratch_shapes=[
            pltpu.VMEM((NUM_GRAPHS, HID), jnp.float32),
            pltpu.VMEM((NUM_GRAPHS, 1), jnp.float32),
        ],
    )(p, batch3, bias)


def _make_edge_pass_split(ch):
    dh = 64
    R = dh + 16
    nchunk = EPAD // (16 * ch)
    rows_per_tile = NPAD // 16
    nzcopy = rows_per_tile // ch
    mesh = plsc.VectorSubcoreMesh(core_axis_name="c", subcore_axis_name="s")

    @functools.partial(
        pl.kernel, mesh=mesh,
        compiler_params=pltpu.CompilerParams(use_tc_tiling_on_sc=False),
        out_type=jax.ShapeDtypeStruct((2, NPAD, R), jnp.float32),
        scratch_types=[
            [pltpu.VMEM((ch,), jnp.int32)] * IDEPTH,
            [pltpu.VMEM((ch,), jnp.int32)] * IDEPTH,
            [pltpu.VMEM((ch, R), jnp.float32)] * DEPTH,
            [pltpu.VMEM((ch, 32), jnp.float32)] * DEPTH,
            pltpu.VMEM_SHARED((NPAD, R), jnp.float32),
            [pltpu.SemaphoreType.DMA] * IDEPTH,
            [pltpu.SemaphoreType.DMA] * DEPTH,
            [pltpu.SemaphoreType.DMA] * DEPTH,
            [pltpu.SemaphoreType.DMA] * DEPTH,
        ],
    )
    def edge_pass(htab, ttab, src_hbm, dst_hbm, out_hbm,
                  src_idx, dst_idx, bufs, tbufs, accum, semi, semg, semt, sems):
        cid = lax.axis_index("c")
        sid = lax.axis_index("s")
        row0 = sid * rows_per_tile
        cbase = sid * nchunk
        srcoff = cid * NPAD

        def zrow(r, carry):
            for j in range(R // 16):
                bufs[0][r, pl.ds(j * 16, 16)] = jnp.zeros((16,), jnp.float32)
            return carry
        lax.fori_loop(0, ch, zrow, 0)
        for j in range(nzcopy):
            pltpu.sync_copy(bufs[0], accum.at[pl.ds(row0 + j * ch, ch)])
        plsc.subcore_barrier()

        def issue_idx(c, bi):
            pltpu.async_copy(src_hbm.at[cbase + c], src_idx[bi], semi[bi])
            pltpu.async_copy(dst_hbm.at[cbase + c], dst_idx[bi], semi[bi])

        def wait_idx(bi):
            pltpu.make_async_copy(src_hbm.at[0], src_idx[bi], semi[bi]).wait()
            pltpu.make_async_copy(dst_hbm.at[0], dst_idx[bi], semi[bi]).wait()
            for j in range(ch // 16):
                src_idx[bi][pl.ds(j * 16, 16)] = (
                    src_idx[bi][pl.ds(j * 16, 16)] + srcoff)

        def issue_gather(b, bi):
            pltpu.async_copy(htab.at[src_idx[bi]], bufs[b], semg[b])
            pltpu.async_copy(ttab.at[dst_idx[bi]], tbufs[b], semt[b])

        def wait_gather(b):
            pltpu.make_async_copy(htab.at[src_idx[0]], bufs[b], semg[b]).wait()
            pltpu.make_async_copy(ttab.at[dst_idx[0]], tbufs[b], semt[b]).wait()

        def issue_scatter(b, bi):
            pltpu.async_copy(bufs[b], accum.at[dst_idx[bi]], sems[b], add=True)

        def wait_scatter(b):
            pltpu.make_async_copy(bufs[b], accum.at[dst_idx[0]], sems[b]).wait()

        def compute(b):
            buf = bufs[b]
            tbuf = tbufs[b]

            def edge(e2, c2):
                for u in range(2):
                    e = e2 * 2 + u
                    tv = jnp.where(cid == 0, tbuf[e, pl.ds(0, 16)],
                                   tbuf[e, pl.ds(16, 16)])
                    s = buf[e, pl.ds(dh, 16)] + tv
                    s = jnp.maximum(s, s * 0.2)
                    w = jnp.exp(s)
                    buf[e, pl.ds(dh, 16)] = w
                    for hd in range(dh // 16):
                        buf[e, pl.ds(hd * 16, 16)] = (
                            buf[e, pl.ds(hd * 16, 16)] * w[hd])
                return c2
            lax.fori_loop(0, ch // 2, edge, 0)

        issue_idx(0, 0)
        issue_idx(1, 1)
        issue_idx(2, 2)
        wait_idx(0)
        issue_gather(0, 0)
        wait_idx(1)
        issue_gather(1, 1)

        def outer(g, carry):
            for u in range(IDEPTH):
                c = g * IDEPTH + u
                b = u % DEPTH
                wait_gather(b)
                compute(b)
                issue_scatter(b, u)

                @pl.when(c >= 1)
                def _():
                    wait_scatter((b + 2) % DEPTH)

                @pl.when(c + 2 < nchunk)
                def _():
                    wait_idx((u + 2) % IDEPTH)
                    issue_gather((b + 2) % DEPTH, (u + 2) % IDEPTH)

                @pl.when(c + 3 < nchunk)
                def _():
                    issue_idx(c + 3, (u + 3) % IDEPTH)
            return carry
        lax.fori_loop(0, nchunk // IDEPTH, outer, 0)

        wait_scatter((nchunk - 1) % DEPTH)
        plsc.subcore_barrier()
        pltpu.sync_copy(accum.at[pl.ds(row0, rows_per_tile)],
                        out_hbm.at[cid, pl.ds(row0, rows_per_tile)])

    return edge_pass


def _make_edge_pass(dh, ch):
    R = dh + 16
    nchunk = EPAD // (NW * ch)
    rows_per_tile = NPAD // 16
    nzcopy = rows_per_tile // ch
    mesh = plsc.VectorSubcoreMesh(core_axis_name="c", subcore_axis_name="s")

    @functools.partial(
        pl.kernel, mesh=mesh,
        compiler_params=pltpu.CompilerParams(use_tc_tiling_on_sc=False),
        out_type=jax.ShapeDtypeStruct((2, NPAD, R), jnp.float32),
        scratch_types=[
            [pltpu.VMEM((ch,), jnp.int32)] * IDEPTH,
            [pltpu.VMEM((ch,), jnp.int32)] * IDEPTH,
            [pltpu.VMEM((ch, R), jnp.float32)] * DEPTH,
            [pltpu.VMEM((ch, 16), jnp.float32)] * DEPTH,
            pltpu.VMEM_SHARED((NPAD, R), jnp.float32),
            [pltpu.SemaphoreType.DMA] * IDEPTH,
            [pltpu.SemaphoreType.DMA] * DEPTH,
            [pltpu.SemaphoreType.DMA] * DEPTH,
            [pltpu.SemaphoreType.DMA] * DEPTH,
        ],
    )
    def edge_pass(htab, ttab, src_hbm, dst_hbm, out_hbm,
                  src_idx, dst_idx, bufs, tbufs, accum, semi, semg, semt, sems):
        cid = lax.axis_index("c")
        sid = lax.axis_index("s")
        wid = cid * 16 + sid
        row0 = sid * rows_per_tile
        cbase = wid * nchunk

        def zrow(r, carry):
            for j in range(R // 16):
                bufs[0][r, pl.ds(j * 16, 16)] = jnp.zeros((16,), jnp.float32)
            return carry
        lax.fori_loop(0, ch, zrow, 0)
        for j in range(nzcopy):
            pltpu.sync_copy(bufs[0], accum.at[pl.ds(row0 + j * ch, ch)])
        plsc.subcore_barrier()

        def issue_idx(c, bi):
            pltpu.async_copy(src_hbm.at[cbase + c], src_idx[bi], semi[bi])
            pltpu.async_copy(dst_hbm.at[cbase + c], dst_idx[bi], semi[bi])

        def wait_idx(bi):
            pltpu.make_async_copy(src_hbm.at[0], src_idx[bi], semi[bi]).wait()
            pltpu.make_async_copy(dst_hbm.at[0], dst_idx[bi], semi[bi]).wait()

        def issue_gather(b, bi):
            pltpu.async_copy(htab.at[src_idx[bi]], bufs[b], semg[b])
            pltpu.async_copy(ttab.at[dst_idx[bi]], tbufs[b], semt[b])

        def wait_gather(b):
            pltpu.make_async_copy(htab.at[src_idx[0]], bufs[b], semg[b]).wait()
            pltpu.make_async_copy(ttab.at[dst_idx[0]], tbufs[b], semt[b]).wait()

        def issue_scatter(b, bi):
            pltpu.async_copy(bufs[b], accum.at[dst_idx[bi]], sems[b], add=True)

        def wait_scatter(b):
            pltpu.make_async_copy(bufs[b], accum.at[dst_idx[0]], sems[b]).wait()

        def compute(b):
            buf = bufs[b]
            tbuf = tbufs[b]

            def edge(e2, c2):
                for u in range(2):
                    e = e2 * 2 + u
                    s = buf[e, pl.ds(dh, 16)] + tbuf[e, :]
                    s = jnp.maximum(s, s * 0.2)
                    w = jnp.exp(s)
                    buf[e, pl.ds(dh, 16)] = w
                    for hd in range(dh // 16):
                        buf[e, pl.ds(hd * 16, 16)] = (
                            buf[e, pl.ds(hd * 16, 16)] * w[hd])
                return c2
            lax.fori_loop(0, ch // 2, edge, 0)

        issue_idx(0, 0)
        issue_idx(1, 1)
        issue_idx(2, 2)
        wait_idx(0)
        issue_gather(0, 0)
        wait_idx(1)
        issue_gather(1, 1)

        def outer(g, carry):
            for u in range(IDEPTH):
                c = g * IDEPTH + u
                b = u % DEPTH
                wait_gather(b)
                compute(b)
                issue_scatter(b, u)

                @pl.when(c >= 1)
                def _():
                    wait_scatter((b + 2) % DEPTH)

                @pl.when(c + 2 < nchunk)
                def _():
                    wait_idx((u + 2) % IDEPTH)
                    issue_gather((b + 2) % DEPTH, (u + 2) % IDEPTH)

                @pl.when(c + 3 < nchunk)
                def _():
                    issue_idx(c + 3, (u + 3) % IDEPTH)
            return carry
        lax.fori_loop(0, nchunk // IDEPTH, outer, 0)

        wait_scatter((nchunk - 1) % DEPTH)
        plsc.subcore_barrier()
        pltpu.sync_copy(accum.at[pl.ds(row0, rows_per_tile)],
                        out_hbm.at[cid, pl.ds(row0, rows_per_tile)])

    return edge_pass


CH = 128
_edge_pass_1 = _make_edge_pass_split(CH)
_edge_pass_2 = _make_edge_pass(HID, CH)


def kernel(x, edge_index, edge_attr, batch, W1, a_src1, a_dst1, b1,
           W2, a_src2, a_dst2, b2):
    del edge_attr
    W1r = W1.reshape(IN_CH, HEADS, HID)
    ws1 = jnp.einsum('ihc,hc->ih', W1r, a_src1)
    wd1 = jnp.einsum('ihc,hc->ih', W1r, a_dst1)
    B1a = jnp.stack([
        jnp.concatenate([W1[:, :64], jnp.tile(ws1[:, :4], (1, 4))], axis=1),
        jnp.concatenate([W1[:, 64:], jnp.tile(ws1[:, 4:], (1, 4))], axis=1),
    ])
    B1b = jnp.concatenate([jnp.tile(wd1[:, :4], (1, 4)),
                           jnp.tile(wd1[:, 4:], (1, 4))], axis=1)
    ws2 = (W2 @ a_src2[0])[:, None]
    wd2 = (W2 @ a_dst2[0])[:, None]
    B2a = jnp.concatenate([W2, jnp.tile(ws2, (1, HID))], axis=1)
    B2b = jnp.tile(wd2, (1, HID))

    xp = jnp.zeros((NPAD, IN_CH), jnp.float32).at[:N].set(x)
    idx_dtype = edge_index.dtype
    loop = jnp.arange(N, dtype=idx_dtype)
    fake = jnp.full((EPAD - E - N,), N, dtype=idx_dtype)
    src = jnp.concatenate([edge_index[0], loop, fake]).astype(jnp.int32)
    dst = jnp.concatenate([edge_index[1], loop, fake]).astype(jnp.int32)

    batch_pad = jnp.concatenate(
        [batch.astype(jnp.int32),
         jnp.full((NPAD - N,), NUM_GRAPHS, jnp.int32)]).reshape(GRID, 1, BLK)

    src2d = src.reshape(-1, CH)
    dst2d = dst.reshape(-1, CH)
    hst, t1 = _mm_l1(xp, B1a, B1b)
    p1 = _edge_pass_1(hst.reshape(2 * NPAD, 64 + HID), t1, src2d, dst2d)
    h2, t2 = _norm_mm2(p1, b1.reshape(1, IN_CH), B2a, B2b)
    p2 = _edge_pass_2(h2, t2, src2d, dst2d)
    return _pool(p2, batch_pad, b2.reshape(1, HID))
